# baseline (speedup 1.0000x reference)
import functools

import jax
import jax.numpy as jnp
from jax import lax
from jax.experimental import pallas as pl
from jax.experimental.pallas import tpu as pltpu
from jax.experimental.pallas import tpu_sc as plsc

VOCAB = 1000000
EMBED = 32
SEQ = 50
NROWS = 16384
NUM_CORES = 2
NUM_SUBCORES = 16
E_PER_CORE = EMBED // NUM_CORES
I_PER_TILE = NROWS // NUM_SUBCORES
IDX_PER_TILE = SEQ * I_PER_TILE
J_GROUP = 10
N_GROUPS = SEQ // J_GROUP
G_ELEMS = J_GROUP * I_PER_TILE
W_CHUNK = 62464
W_TAIL_OFF = 16 * W_CHUNK
W_TAIL_PAD = 1024
VOCAB_PAD = W_TAIL_OFF + W_TAIL_PAD
_mesh = plsc.VectorSubcoreMesh(core_axis_name="c", subcore_axis_name="s")


@functools.partial(
    pl.kernel,
    out_type=jax.ShapeDtypeStruct((SEQ, EMBED, NROWS), jnp.float32),
    mesh=_mesh,
    scratch_types=[
        pltpu.VMEM((IDX_PER_TILE,), jnp.int32),
        pltpu.VMEM((G_ELEMS,), jnp.float32),
        pltpu.VMEM_SHARED((VOCAB_PAD,), jnp.float32),
        pltpu.SemaphoreType.DMA,
        pltpu.SemaphoreType.DMA,
    ],
    compiler_params=pltpu.CompilerParams(use_tc_tiling_on_sc=True),
)
def _gather_kernel(
    xt_hbm, wt_hbm, wt_tail_hbm, out_hbm, idx_v, rows_v, w_sh, sem, sem_o
):
    c = lax.axis_index("c")
    s = lax.axis_index("s")
    i0 = s * I_PER_TILE

    def stage_j(j, _):
        pltpu.sync_copy(
            xt_hbm.at[j, pl.ds(i0, I_PER_TILE)],
            idx_v.at[pl.ds(j * I_PER_TILE, I_PER_TILE)],
        )
        return ()

    lax.fori_loop(0, SEQ, stage_j, ())

    def e_body(eo, _):
        e = c * E_PER_CORE + eo
        plsc.subcore_barrier()

        pltpu.sync_copy(
            wt_hbm.at[e, pl.ds(s * W_CHUNK, W_CHUNK)],
            w_sh.at[pl.ds(s * W_CHUNK, W_CHUNK)],
        )

        @pl.when(s == 0)
        def _load_tail():
            pltpu.sync_copy(
                wt_tail_hbm.at[e], w_sh.at[pl.ds(W_TAIL_OFF, W_TAIL_PAD)]
            )

        plsc.subcore_barrier()

        def group_body(g, _):
            pltpu.async_copy(
                w_sh.at[idx_v.at[pl.ds(g * G_ELEMS, G_ELEMS)]], rows_v, sem
            ).wait()

            def out_j(j, _):
                pltpu.async_copy(
                    rows_v.at[pl.ds(j * I_PER_TILE, I_PER_TILE)],
                    out_hbm.at[g * J_GROUP + j, e, pl.ds(i0, I_PER_TILE)],
                    sem_o,
                )
                return ()

            lax.fori_loop(0, J_GROUP, out_j, ())

            def drain_j(j, _):
                pltpu.make_async_copy(
                    rows_v.at[pl.ds(j * I_PER_TILE, I_PER_TILE)],
                    out_hbm.at[g * J_GROUP + j, e, pl.ds(i0, I_PER_TILE)],
                    sem_o,
                ).wait()
                return ()

            lax.fori_loop(0, J_GROUP, drain_j, ())
            return ()

        lax.fori_loop(0, N_GROUPS, group_body, ())
        return ()

    lax.fori_loop(0, E_PER_CORE, e_body, ())


def kernel(x, W):
    wt = W.T
    wt_tail = jnp.pad(wt[:, W_TAIL_OFF:], ((0, 0), (0, W_TAIL_PAD - (VOCAB - W_TAIL_OFF))))
    out_t = _gather_kernel(x.T.astype(jnp.int32), wt, wt_tail)
    return jnp.transpose(out_t, (2, 0, 1))


# speedup vs baseline: 1.0579x; 1.0579x over previous
import functools

import jax
import jax.numpy as jnp
from jax import lax
from jax.experimental import pallas as pl
from jax.experimental.pallas import tpu as pltpu
from jax.experimental.pallas import tpu_sc as plsc

VOCAB = 1000000
EMBED = 32
SEQ = 50
NROWS = 16384
NUM_CORES = 2
NUM_SUBCORES = 16
E_PER_CORE = EMBED // NUM_CORES
I_PER_TILE = NROWS // NUM_SUBCORES
IDX_PER_TILE = SEQ * I_PER_TILE
J_GROUP = 5
N_GROUPS = SEQ // J_GROUP
G_ELEMS = J_GROUP * I_PER_TILE
W_CHUNK = 62464
W_TAIL_OFF = 16 * W_CHUNK
W_TAIL_PAD = 1024
VOCAB_PAD = W_TAIL_OFF + W_TAIL_PAD
_mesh = plsc.VectorSubcoreMesh(core_axis_name="c", subcore_axis_name="s")


@functools.partial(
    pl.kernel,
    out_type=jax.ShapeDtypeStruct((SEQ, EMBED, NROWS), jnp.float32),
    mesh=_mesh,
    scratch_types=[
        pltpu.VMEM((IDX_PER_TILE,), jnp.int32),
        pltpu.VMEM((G_ELEMS,), jnp.float32),
        pltpu.VMEM((G_ELEMS,), jnp.float32),
        pltpu.VMEM_SHARED((VOCAB_PAD,), jnp.float32),
        pltpu.SemaphoreType.DMA,
        pltpu.SemaphoreType.DMA,
        pltpu.SemaphoreType.DMA,
        pltpu.SemaphoreType.DMA,
    ],
    compiler_params=pltpu.CompilerParams(use_tc_tiling_on_sc=True),
)
def _gather_kernel(
    xt_hbm,
    wt_hbm,
    wt_tail_hbm,
    out_hbm,
    idx_v,
    rows_a,
    rows_b,
    w_sh,
    sem_a,
    sem_b,
    semo_a,
    semo_b,
):
    c = lax.axis_index("c")
    s = lax.axis_index("s")
    i0 = s * I_PER_TILE

    def stage_j(j, _):
        pltpu.sync_copy(
            xt_hbm.at[j, pl.ds(i0, I_PER_TILE)],
            idx_v.at[pl.ds(j * I_PER_TILE, I_PER_TILE)],
        )
        return ()

    lax.fori_loop(0, SEQ, stage_j, ())

    def e_body(eo, _):
        e = c * E_PER_CORE + eo
        plsc.subcore_barrier()

        pltpu.sync_copy(
            wt_hbm.at[e, pl.ds(s * W_CHUNK, W_CHUNK)],
            w_sh.at[pl.ds(s * W_CHUNK, W_CHUNK)],
        )

        @pl.when(s == 0)
        def _load_tail():
            pltpu.sync_copy(
                wt_tail_hbm.at[e], w_sh.at[pl.ds(W_TAIL_OFF, W_TAIL_PAD)]
            )

        plsc.subcore_barrier()

        bufs = (rows_a, rows_b)
        gsems = (sem_a, sem_b)
        osems = (semo_a, semo_b)

        def gather_desc(g):
            b = g % 2
            return pltpu.make_async_copy(
                w_sh.at[idx_v.at[pl.ds(g * G_ELEMS, G_ELEMS)]],
                bufs[b],
                gsems[b],
            )

        def write_descs(g):
            b = g % 2
            return [
                pltpu.make_async_copy(
                    bufs[b].at[pl.ds(j * I_PER_TILE, I_PER_TILE)],
                    out_hbm.at[g * J_GROUP + j, e, pl.ds(i0, I_PER_TILE)],
                    osems[b],
                )
                for j in range(J_GROUP)
            ]

        gather_desc(0).start()
        for g in range(N_GROUPS):
            if g + 1 < N_GROUPS:
                if g >= 1:
                    for d in write_descs(g - 1):
                        d.wait()
                gather_desc(g + 1).start()
            gather_desc(g).wait()
            for d in write_descs(g):
                d.start()
        for d in write_descs(N_GROUPS - 2):
            d.wait()
        for d in write_descs(N_GROUPS - 1):
            d.wait()
        return ()

    lax.fori_loop(0, E_PER_CORE, e_body, ())


def kernel(x, W):
    wt = W.T
    wt_tail = jnp.pad(wt[:, W_TAIL_OFF:], ((0, 0), (0, W_TAIL_PAD - (VOCAB - W_TAIL_OFF))))
    out_t = _gather_kernel(x.T.astype(jnp.int32), wt, wt_tail)
    return jnp.transpose(out_t, (2, 0, 1))


# speedup vs baseline: 1.1665x; 1.1027x over previous
import functools

import jax
import jax.numpy as jnp
from jax import lax
from jax.experimental import pallas as pl
from jax.experimental.pallas import tpu as pltpu
from jax.experimental.pallas import tpu_sc as plsc

VOCAB = 1000000
EMBED = 32
SEQ = 50
NROWS = 16384
NUM_CORES = 2
NUM_SUBCORES = 16
E_PER_CORE = EMBED // NUM_CORES
I_PER_TILE = NROWS // NUM_SUBCORES
IDX_PER_TILE = SEQ * I_PER_TILE
J_GROUP = 5
N_GROUPS = SEQ // J_GROUP
G_ELEMS = J_GROUP * I_PER_TILE
_mesh = plsc.VectorSubcoreMesh(core_axis_name="c", subcore_axis_name="s")


@functools.partial(
    pl.kernel,
    out_type=jax.ShapeDtypeStruct((SEQ, EMBED, NROWS), jnp.float32),
    mesh=_mesh,
    scratch_types=[
        pltpu.VMEM((IDX_PER_TILE,), jnp.int32),
        pltpu.VMEM((G_ELEMS,), jnp.float32),
        pltpu.VMEM((G_ELEMS,), jnp.float32),
        pltpu.VMEM_SHARED((VOCAB,), jnp.float32),
        pltpu.SemaphoreType.DMA,
        pltpu.SemaphoreType.DMA,
        pltpu.SemaphoreType.DMA,
        pltpu.SemaphoreType.DMA,
    ],
    compiler_params=pltpu.CompilerParams(use_tc_tiling_on_sc=True),
)
def _gather_kernel(
    xt_hbm,
    wt_hbm,
    out_hbm,
    idx_v,
    rows_a,
    rows_b,
    w_sh,
    sem_a,
    sem_b,
    semo_a,
    semo_b,
):
    c = lax.axis_index("c")
    s = lax.axis_index("s")
    i0 = s * I_PER_TILE

    def _idx_desc(j):
        return pltpu.make_async_copy(
            xt_hbm.at[j, pl.ds(i0, I_PER_TILE)],
            idx_v.at[pl.ds(j * I_PER_TILE, I_PER_TILE)],
            sem_a,
        )

    def stage_j(j, _):
        _idx_desc(j).start()
        return ()

    def drain_idx(j, _):
        _idx_desc(j).wait()
        return ()

    lax.fori_loop(0, SEQ, stage_j, ())
    lax.fori_loop(0, SEQ, drain_idx, ())

    def e_body(eo, _):
        e = c * E_PER_CORE + eo
        plsc.subcore_barrier()

        @pl.when(s == 0)
        def _load_row():
            pltpu.sync_copy(wt_hbm.at[e], w_sh)

        plsc.subcore_barrier()

        bufs = (rows_a, rows_b)
        gsems = (sem_a, sem_b)
        osems = (semo_a, semo_b)

        def gather_desc(g):
            b = g % 2
            return pltpu.make_async_copy(
                w_sh.at[idx_v.at[pl.ds(g * G_ELEMS, G_ELEMS)]],
                bufs[b],
                gsems[b],
            )

        def write_descs(g):
            b = g % 2
            return [
                pltpu.make_async_copy(
                    bufs[b].at[pl.ds(j * I_PER_TILE, I_PER_TILE)],
                    out_hbm.at[g * J_GROUP + j, e, pl.ds(i0, I_PER_TILE)],
                    osems[b],
                )
                for j in range(J_GROUP)
            ]

        gather_desc(0).start()
        for g in range(N_GROUPS):
            if g + 1 < N_GROUPS:
                if g >= 1:
                    for d in write_descs(g - 1):
                        d.wait()
                gather_desc(g + 1).start()
            gather_desc(g).wait()
            for d in write_descs(g):
                d.start()
        for d in write_descs(N_GROUPS - 2):
            d.wait()
        for d in write_descs(N_GROUPS - 1):
            d.wait()
        return ()

    lax.fori_loop(0, E_PER_CORE, e_body, ())


def kernel(x, W):
    out_t = _gather_kernel(x.T.astype(jnp.int32), W.T)
    return jnp.transpose(out_t, (2, 0, 1))
